# baseline (speedup 1.0000x reference)
import jax
import jax.numpy as jnp
from jax import lax
from jax.experimental import pallas as pl
from jax.experimental.pallas import tpu as pltpu
from jax.experimental.pallas import tpu_sc as plsc

N = 10000
E = 320000
D = 128
H = 8
DK = 16

NC = 2
NS = 16
NW = NC * NS
C = 32
NCHUNK = 316
EP = NCHUNK * C
EPAD = NW * EP
TRASH = 10100
ROW = D + 2 * H
NP = 10240
RPT = NP // NS


def _proj_body(x_ref, w_ref, q_ref, kv_ref):
    full = jnp.dot(x_ref[...], w_ref[...], preferred_element_type=jnp.float32)
    q_ref[...] = full[:, :D]
    kv_ref[...] = full[:, D:]


def _project(x, w):
    blk = 1000
    return pl.pallas_call(
        _proj_body,
        grid=(N // blk,),
        in_specs=[
            pl.BlockSpec((blk, D), lambda i: (i, 0)),
            pl.BlockSpec((D, 3 * D), lambda i: (0, 0)),
        ],
        out_specs=[
            pl.BlockSpec((blk, D), lambda i: (i, 0)),
            pl.BlockSpec((blk, 2 * D), lambda i: (i, 0)),
        ],
        out_shape=[
            jax.ShapeDtypeStruct((N, D), jnp.float32),
            jax.ShapeDtypeStruct((N, 2 * D), jnp.float32),
        ],
    )(x, w)


def _edge_body(q_hbm, kv_hbm, src_hbm, dst_hbm, part_hbm, acc_sp,
               src0, src1, dst0, dst1, dsts0, dsts1,
               kv0, kv1, q0, q1, out0, out1,
               si0, si1, skv0, skv1, sq0, sq1, ss0, ss1):
    core = lax.axis_index("c")
    sub = lax.axis_index("s")
    wid = sub * NC + core

    buf = [(src0, dst0, dsts0, kv0, q0, out0, si0, skv0, sq0, ss0),
           (src1, dst1, dsts1, kv1, q1, out1, si1, skv1, sq1, ss1)]

    def _zrow(r, carry):
        for c16 in range(ROW // 16):
            out0[r, pl.ds(c16 * 16, 16)] = jnp.zeros((16,), jnp.float32)
        return carry
    lax.fori_loop(0, C, _zrow, 0)
    for j in range(RPT // C):
        pltpu.sync_copy(out0, acc_sp.at[pl.ds(sub * RPT + j * C, C)])

    plsc.subcore_barrier()

    iota = lax.broadcasted_iota(jnp.int32, (16,), 0)
    last = jnp.full((16,), DK - 1, jnp.int32)
    hmask = [(iota == h).astype(jnp.float32) for h in range(H)]

    def _idx_start(c, b):
        base = wid * EP + c * C
        pltpu.async_copy(src_hbm.at[pl.ds(base, C)], b[0], b[6])
        pltpu.async_copy(dst_hbm.at[pl.ds(base, C)], b[1], b[6])

    def _idx_wait(b):
        pltpu.make_async_copy(src_hbm.at[pl.ds(0, C)], b[0], b[6]).wait()
        pltpu.make_async_copy(dst_hbm.at[pl.ds(0, C)], b[1], b[6]).wait()

    def _gathers_start(b):
        pltpu.async_copy(kv_hbm.at[b[0]], b[3], b[7])
        pltpu.async_copy(q_hbm.at[b[1]], b[4], b[8])

    def _gathers_wait(b):
        pltpu.make_async_copy(kv_hbm.at[b[0]], b[3], b[7]).wait()
        pltpu.make_async_copy(q_hbm.at[b[1]], b[4], b[8]).wait()

    def _dsts_copy(b):
        for r in range(C // 16):
            b[2][pl.ds(r * 16, 16)] = b[1][pl.ds(r * 16, 16)]

    def _scatter_start(b):
        pltpu.async_copy(b[5], acc_sp.at[b[2]], b[9], add=True)

    def _scatter_wait(b):
        pltpu.make_async_copy(b[5], acc_sp.at[b[2]], b[9]).wait()

    def _compute(b):
        kv_v, q_v, out_v = b[3], b[4], b[5]

        UNROLL = 2

        def _edges(i, carry):
            es = [i * UNROLL + u for u in range(UNROLL)]
            tv = [[None] * H for _ in es]
            for h in range(H):
                for u, e in enumerate(es):
                    kr = kv_v[e, pl.ds(h * DK, DK)]
                    qr = q_v[e, pl.ds(h * DK, DK)]
                    tv[u][h] = jnp.take(jnp.cumsum(kr * qr), last) * hmask[h]
            pv = []
            for u, e in enumerate(es):
                t = tv[u]
                s = (((t[0] + t[1]) + (t[2] + t[3]))
                     + ((t[4] + t[5]) + (t[6] + t[7])))
                p = jnp.exp(jnp.minimum(jnp.maximum(s, -10.0), 10.0))
                pv.append(p)
                out_v[e, pl.ds(D, 16)] = p
            for h in range(H):
                hh = jnp.full((16,), h, jnp.int32)
                for u, e in enumerate(es):
                    vr = kv_v[e, pl.ds(D + h * DK, DK)]
                    out_v[e, pl.ds(h * DK, DK)] = vr * jnp.take(pv[u], hh)
            return carry
        lax.fori_loop(0, C // UNROLL, _edges, 0)

    NPAIR = NCHUNK // 2
    _idx_start(jnp.int32(0), buf[0])
    _idx_start(jnp.int32(1), buf[1])
    _idx_wait(buf[0])
    _gathers_start(buf[0])

    def _pair(j, carry):
        for s in range(2):
            c = 2 * j + s
            b, o = buf[s], buf[1 - s]
            _gathers_wait(b)

            @pl.when(j > 0)
            def _():
                _scatter_wait(b)

            _dsts_copy(b)

            @pl.when(j < NPAIR - 1)
            def _():
                _idx_start(c + 2, b)

            if s == 0:
                _idx_wait(o)
                _gathers_start(o)
            else:
                @pl.when(j < NPAIR - 1)
                def _():
                    _idx_wait(o)
                    _gathers_start(o)

            _compute(b)
            _scatter_start(b)
        return carry

    lax.fori_loop(0, NPAIR, _pair, 0)
    _scatter_wait(buf[0])
    _scatter_wait(buf[1])

    plsc.subcore_barrier()

    pltpu.sync_copy(acc_sp.at[pl.ds(sub * RPT, RPT)],
                    part_hbm.at[core, pl.ds(sub * RPT, RPT)])


def _edge_attention(q_tab, kv_tab, src, dst):
    mesh = plsc.VectorSubcoreMesh(core_axis_name="c", subcore_axis_name="s")
    return pl.kernel(
        _edge_body,
        out_type=jax.ShapeDtypeStruct((NC, NP, ROW), jnp.float32),
        mesh=mesh,
        compiler_params=pltpu.CompilerParams(
            use_tc_tiling_on_sc=False, needs_layout_passes=False),
        scratch_types=[
            pltpu.VMEM_SHARED((NP, ROW), jnp.float32),
            pltpu.VMEM((C,), jnp.int32),
            pltpu.VMEM((C,), jnp.int32),
            pltpu.VMEM((C,), jnp.int32),
            pltpu.VMEM((C,), jnp.int32),
            pltpu.VMEM((C,), jnp.int32),
            pltpu.VMEM((C,), jnp.int32),
            pltpu.VMEM((C, 2 * D), jnp.float32),
            pltpu.VMEM((C, 2 * D), jnp.float32),
            pltpu.VMEM((C, D), jnp.float32),
            pltpu.VMEM((C, D), jnp.float32),
            pltpu.VMEM((C, ROW), jnp.float32),
            pltpu.VMEM((C, ROW), jnp.float32),
        ] + [pltpu.SemaphoreType.DMA] * 8,
    )(q_tab, kv_tab, src, dst)


def _out_body(part_ref, wo_ref, o_ref):
    both = part_ref[...]
    tot = both[0] + both[1]
    wv = tot[:, :D]
    z = tot[:, D:D + H]
    rows = lax.broadcasted_iota(jnp.int32, (H, D), 0)
    cols = lax.broadcasted_iota(jnp.int32, (H, D), 1)
    sel = (cols // DK == rows).astype(jnp.float32)
    norm = jnp.dot(z, sel, preferred_element_type=jnp.float32) + 1e-6
    o_ref[...] = jnp.dot(wv / norm, wo_ref[...],
                         preferred_element_type=jnp.float32)


def _finalize(part, wo):
    blk = 1000
    return pl.pallas_call(
        _out_body,
        grid=(N // blk,),
        in_specs=[
            pl.BlockSpec((NC, blk, ROW), lambda i: (0, i, 0)),
            pl.BlockSpec((D, D), lambda i: (0, 0)),
        ],
        out_specs=pl.BlockSpec((blk, D), lambda i: (i, 0)),
        out_shape=jax.ShapeDtypeStruct((N, D), jnp.float32),
    )(part, wo)


@jax.jit
def kernel(x, edge_index, Wq, Wk, Wv, Wo):
    w = jnp.concatenate([Wq, Wk * 0.25, Wv], axis=1)
    q_tab, kv_tab = _project(x, w)
    src = edge_index[0].astype(jnp.int32)
    dst = edge_index[1].astype(jnp.int32)
    pad = EPAD - E
    src_p = jnp.concatenate([src, jnp.zeros((pad,), jnp.int32)])
    dst_p = jnp.concatenate([dst, jnp.full((pad,), TRASH, jnp.int32)])
    part = _edge_attention(q_tab, kv_tab, src_p, dst_p)
    return _finalize(part, Wo)


# speedup vs baseline: 1.0220x; 1.0220x over previous
import jax
import jax.numpy as jnp
from jax import lax
from jax.experimental import pallas as pl
from jax.experimental.pallas import tpu as pltpu
from jax.experimental.pallas import tpu_sc as plsc

N = 10000
E = 320000
D = 128
H = 8
DK = 16

NC = 2
NS = 16
NW = NC * NS
C = 32
NCHUNK = 316
EP = NCHUNK * C
EPAD = NW * EP
TRASH = 10100
ROW = D + 2 * H
NP = 10240
RPT = NP // NS


def _proj_body(x_ref, w_ref, q_ref, kv_ref):
    full = jnp.dot(x_ref[...], w_ref[...], preferred_element_type=jnp.float32)
    q_ref[...] = full[:, :D]
    kv_ref[...] = full[:, D:]


def _project(x, w):
    blk = 1000
    return pl.pallas_call(
        _proj_body,
        grid=(N // blk,),
        in_specs=[
            pl.BlockSpec((blk, D), lambda i: (i, 0)),
            pl.BlockSpec((D, 3 * D), lambda i: (0, 0)),
        ],
        out_specs=[
            pl.BlockSpec((blk, D), lambda i: (i, 0)),
            pl.BlockSpec((blk, 2 * D), lambda i: (i, 0)),
        ],
        out_shape=[
            jax.ShapeDtypeStruct((N, D), jnp.float32),
            jax.ShapeDtypeStruct((N, 2 * D), jnp.float32),
        ],
    )(x, w)


def _edge_body(q_hbm, kv_hbm, src_hbm, dst_hbm, part_hbm, acc_sp,
               src0, src1, dst0, dst1, dsts0, dsts1,
               kv0, kv1, q0, q1, out0, out1,
               si0, si1, skv0, skv1, sq0, sq1, ss0, ss1):
    core = lax.axis_index("c")
    sub = lax.axis_index("s")
    wid = sub * NC + core

    buf = [(src0, dst0, dsts0, kv0, q0, out0, si0, skv0, sq0, ss0),
           (src1, dst1, dsts1, kv1, q1, out1, si1, skv1, sq1, ss1)]

    def _zrow(r, carry):
        for c16 in range(ROW // 16):
            out0[r, pl.ds(c16 * 16, 16)] = jnp.zeros((16,), jnp.float32)
        return carry
    lax.fori_loop(0, C, _zrow, 0)
    for j in range(RPT // C):
        pltpu.sync_copy(out0, acc_sp.at[pl.ds(sub * RPT + j * C, C)])

    plsc.subcore_barrier()

    iota = lax.broadcasted_iota(jnp.int32, (16,), 0)
    last = jnp.full((16,), DK - 1, jnp.int32)
    hone = [iota == h for h in range(H)]

    def _idx_start(c, b):
        base = wid * EP + c * C
        pltpu.async_copy(src_hbm.at[pl.ds(base, C)], b[0], b[6])
        pltpu.async_copy(dst_hbm.at[pl.ds(base, C)], b[1], b[6])

    def _idx_wait(b):
        pltpu.make_async_copy(src_hbm.at[pl.ds(0, C)], b[0], b[6]).wait()
        pltpu.make_async_copy(dst_hbm.at[pl.ds(0, C)], b[1], b[6]).wait()

    def _gathers_start(b):
        pltpu.async_copy(kv_hbm.at[b[0]], b[3], b[7])
        pltpu.async_copy(q_hbm.at[b[1]], b[4], b[8])

    def _gathers_wait(b):
        pltpu.make_async_copy(kv_hbm.at[b[0]], b[3], b[7]).wait()
        pltpu.make_async_copy(q_hbm.at[b[1]], b[4], b[8]).wait()

    def _dsts_copy(b):
        for r in range(C // 16):
            b[2][pl.ds(r * 16, 16)] = b[1][pl.ds(r * 16, 16)]

    def _scatter_start(b):
        pltpu.async_copy(b[5], acc_sp.at[b[2]], b[9], add=True)

    def _scatter_wait(b):
        pltpu.make_async_copy(b[5], acc_sp.at[b[2]], b[9]).wait()

    def _compute(b):
        kv_v, q_v, out_v = b[3], b[4], b[5]

        UNROLL = 2

        def _edges(i, carry):
            es = [i * UNROLL + u for u in range(UNROLL)]
            sv = [jnp.zeros((16,), jnp.float32) for _ in es]
            for h in range(H):
                for u, e in enumerate(es):
                    kr = kv_v[e, pl.ds(h * DK, DK)]
                    qr = q_v[e, pl.ds(h * DK, DK)]
                    tot = jnp.take(jnp.cumsum(kr * qr), last)
                    sv[u] = jnp.where(hone[h], tot, sv[u])
            pv = []
            for u, e in enumerate(es):
                p = jnp.exp(jnp.minimum(jnp.maximum(sv[u], -10.0), 10.0))
                pv.append(p)
                out_v[e, pl.ds(D, 16)] = p
            for h in range(H):
                hh = jnp.full((16,), h, jnp.int32)
                for u, e in enumerate(es):
                    vr = kv_v[e, pl.ds(D + h * DK, DK)]
                    out_v[e, pl.ds(h * DK, DK)] = vr * jnp.take(pv[u], hh)
            return carry
        lax.fori_loop(0, C // UNROLL, _edges, 0)

    NPAIR = NCHUNK // 2
    _idx_start(jnp.int32(0), buf[0])
    _idx_start(jnp.int32(1), buf[1])
    _idx_wait(buf[0])
    _gathers_start(buf[0])

    def _pair(j, carry):
        for s in range(2):
            c = 2 * j + s
            b, o = buf[s], buf[1 - s]
            _gathers_wait(b)

            @pl.when(j > 0)
            def _():
                _scatter_wait(b)

            _dsts_copy(b)

            @pl.when(j < NPAIR - 1)
            def _():
                _idx_start(c + 2, b)

            if s == 0:
                _idx_wait(o)
                _gathers_start(o)
            else:
                @pl.when(j < NPAIR - 1)
                def _():
                    _idx_wait(o)
                    _gathers_start(o)

            _compute(b)
            _scatter_start(b)
        return carry

    lax.fori_loop(0, NPAIR, _pair, 0)
    _scatter_wait(buf[0])
    _scatter_wait(buf[1])

    plsc.subcore_barrier()

    pltpu.sync_copy(acc_sp.at[pl.ds(sub * RPT, RPT)],
                    part_hbm.at[core, pl.ds(sub * RPT, RPT)])


def _edge_attention(q_tab, kv_tab, src, dst):
    mesh = plsc.VectorSubcoreMesh(core_axis_name="c", subcore_axis_name="s")
    return pl.kernel(
        _edge_body,
        out_type=jax.ShapeDtypeStruct((NC, NP, ROW), jnp.float32),
        mesh=mesh,
        compiler_params=pltpu.CompilerParams(
            use_tc_tiling_on_sc=False, needs_layout_passes=False),
        scratch_types=[
            pltpu.VMEM_SHARED((NP, ROW), jnp.float32),
            pltpu.VMEM((C,), jnp.int32),
            pltpu.VMEM((C,), jnp.int32),
            pltpu.VMEM((C,), jnp.int32),
            pltpu.VMEM((C,), jnp.int32),
            pltpu.VMEM((C,), jnp.int32),
            pltpu.VMEM((C,), jnp.int32),
            pltpu.VMEM((C, 2 * D), jnp.float32),
            pltpu.VMEM((C, 2 * D), jnp.float32),
            pltpu.VMEM((C, D), jnp.float32),
            pltpu.VMEM((C, D), jnp.float32),
            pltpu.VMEM((C, ROW), jnp.float32),
            pltpu.VMEM((C, ROW), jnp.float32),
        ] + [pltpu.SemaphoreType.DMA] * 8,
    )(q_tab, kv_tab, src, dst)


def _out_body(part_ref, wo_ref, o_ref):
    both = part_ref[...]
    tot = both[0] + both[1]
    wv = tot[:, :D]
    z = tot[:, D:D + H]
    rows = lax.broadcasted_iota(jnp.int32, (H, D), 0)
    cols = lax.broadcasted_iota(jnp.int32, (H, D), 1)
    sel = (cols // DK == rows).astype(jnp.float32)
    norm = jnp.dot(z, sel, preferred_element_type=jnp.float32) + 1e-6
    o_ref[...] = jnp.dot(wv / norm, wo_ref[...],
                         preferred_element_type=jnp.float32)


def _finalize(part, wo):
    blk = 1000
    return pl.pallas_call(
        _out_body,
        grid=(N // blk,),
        in_specs=[
            pl.BlockSpec((NC, blk, ROW), lambda i: (0, i, 0)),
            pl.BlockSpec((D, D), lambda i: (0, 0)),
        ],
        out_specs=pl.BlockSpec((blk, D), lambda i: (i, 0)),
        out_shape=jax.ShapeDtypeStruct((N, D), jnp.float32),
    )(part, wo)


@jax.jit
def kernel(x, edge_index, Wq, Wk, Wv, Wo):
    w = jnp.concatenate([Wq, Wk * 0.25, Wv], axis=1)
    q_tab, kv_tab = _project(x, w)
    src = edge_index[0].astype(jnp.int32)
    dst = edge_index[1].astype(jnp.int32)
    pad = EPAD - E
    src_p = jnp.concatenate([src, jnp.zeros((pad,), jnp.int32)])
    dst_p = jnp.concatenate([dst, jnp.full((pad,), TRASH, jnp.int32)])
    part = _edge_attention(q_tab, kv_tab, src_p, dst_p)
    return _finalize(part, Wo)


# speedup vs baseline: 1.1893x; 1.1637x over previous
import jax
import jax.numpy as jnp
from jax import lax
from jax.experimental import pallas as pl
from jax.experimental.pallas import tpu as pltpu
from jax.experimental.pallas import tpu_sc as plsc

N = 10000
E = 320000
D = 128
H = 8
DK = 16

NC = 2
NS = 16
NW = NC * NS
C = 32
NCHUNK = 316
EP = NCHUNK * C
EPAD = NW * EP
TRASH = 10100
ROW = D + 2 * H
NP = 10240
RPT = NP // NS


def _proj_body(x_ref, w_ref, q_ref, kv_ref):
    full = jnp.dot(x_ref[...], w_ref[...], preferred_element_type=jnp.float32)
    q_ref[...] = full[:, :D]
    kv_ref[...] = full[:, D:]


def _project(x, w):
    blk = 1000
    return pl.pallas_call(
        _proj_body,
        grid=(N // blk,),
        in_specs=[
            pl.BlockSpec((blk, D), lambda i: (i, 0)),
            pl.BlockSpec((D, 3 * D), lambda i: (0, 0)),
        ],
        out_specs=[
            pl.BlockSpec((blk, D), lambda i: (i, 0)),
            pl.BlockSpec((blk, 2 * D), lambda i: (i, 0)),
        ],
        out_shape=[
            jax.ShapeDtypeStruct((N, D), jnp.float32),
            jax.ShapeDtypeStruct((N, 2 * D), jnp.float32),
        ],
    )(x, w)


def _edge_body(q_hbm, kv_hbm, src_hbm, dst_hbm, part_hbm, acc_sp,
               src0, src1, dst0, dst1, dsts0, dsts1,
               kv0, kv1, q0, q1, out0, out1,
               si0, si1, skv0, skv1, sq0, sq1, ss0, ss1):
    core = lax.axis_index("c")
    sub = lax.axis_index("s")
    wid = sub * NC + core

    buf = [(src0, dst0, dsts0, kv0, q0, out0, si0, skv0, sq0, ss0),
           (src1, dst1, dsts1, kv1, q1, out1, si1, skv1, sq1, ss1)]

    def _zrow(r, carry):
        for c16 in range(ROW // 16):
            out0[r, pl.ds(c16 * 16, 16)] = jnp.zeros((16,), jnp.float32)
        return carry
    lax.fori_loop(0, C, _zrow, 0)
    for j in range(RPT // C):
        pltpu.sync_copy(out0, acc_sp.at[pl.ds(sub * RPT + j * C, C)])

    plsc.subcore_barrier()

    iota = lax.broadcasted_iota(jnp.int32, (16,), 0)
    last = jnp.full((16,), DK - 1, jnp.int32)
    hone = [iota == h for h in range(H)]

    def _idx_start(c, b):
        base = wid * EP + c * C
        pltpu.async_copy(src_hbm.at[pl.ds(base, C)], b[0], b[6])
        pltpu.async_copy(dst_hbm.at[pl.ds(base, C)], b[1], b[6])

    def _idx_wait(b):
        pltpu.make_async_copy(src_hbm.at[pl.ds(0, C)], b[0], b[6]).wait()
        pltpu.make_async_copy(dst_hbm.at[pl.ds(0, C)], b[1], b[6]).wait()

    def _gathers_start(b):
        pltpu.async_copy(kv_hbm.at[b[0]], b[3], b[7])
        pltpu.async_copy(q_hbm.at[b[1]], b[4], b[8])

    def _gathers_wait(b):
        pltpu.make_async_copy(kv_hbm.at[b[0]], b[3], b[7]).wait()
        pltpu.make_async_copy(q_hbm.at[b[1]], b[4], b[8]).wait()

    def _dsts_copy(b):
        for r in range(C // 16):
            b[2][pl.ds(r * 16, 16)] = b[1][pl.ds(r * 16, 16)]

    def _scatter_start(b):
        pltpu.async_copy(b[5], acc_sp.at[b[2]], b[9], add=True)

    def _scatter_wait(b):
        pltpu.make_async_copy(b[5], acc_sp.at[b[2]], b[9]).wait()

    def _compute(b):
        kv_v, q_v, out_v = b[3], b[4], b[5]

        def _edge2(i, carry):
            e0 = i * 2
            e1 = e0 + 1
            s0 = jnp.zeros((16,), jnp.float32)
            s1 = jnp.zeros((16,), jnp.float32)
            for h in range(H):
                k0 = kv_v[e0, pl.ds(h * DK, DK)]
                q0 = q_v[e0, pl.ds(h * DK, DK)]
                k1 = kv_v[e1, pl.ds(h * DK, DK)]
                q1 = q_v[e1, pl.ds(h * DK, DK)]
                t0 = jnp.take(jnp.cumsum(k0 * q0), last)
                t1 = jnp.take(jnp.cumsum(k1 * q1), last)
                s0 = jnp.where(hone[h], t0, s0)
                s1 = jnp.where(hone[h], t1, s1)
            p0 = jnp.exp(jnp.minimum(jnp.maximum(s0, -10.0), 10.0))
            p1 = jnp.exp(jnp.minimum(jnp.maximum(s1, -10.0), 10.0))
            out_v[e0, pl.ds(D, 16)] = p0
            out_v[e1, pl.ds(D, 16)] = p1
            for h in range(H):
                hh = jnp.full((16,), h, jnp.int32)
                v0 = kv_v[e0, pl.ds(D + h * DK, DK)]
                v1 = kv_v[e1, pl.ds(D + h * DK, DK)]
                out_v[e0, pl.ds(h * DK, DK)] = v0 * jnp.take(p0, hh)
                out_v[e1, pl.ds(h * DK, DK)] = v1 * jnp.take(p1, hh)
            return carry
        lax.fori_loop(0, C // 2, _edge2, 0)

    NPAIR = NCHUNK // 2
    _idx_start(jnp.int32(0), buf[0])
    _idx_start(jnp.int32(1), buf[1])
    _idx_wait(buf[0])
    _gathers_start(buf[0])

    def _pair(j, carry):
        for s in range(2):
            c = 2 * j + s
            b, o = buf[s], buf[1 - s]
            _gathers_wait(b)

            @pl.when(j > 0)
            def _():
                _scatter_wait(b)

            _dsts_copy(b)

            @pl.when(j < NPAIR - 1)
            def _():
                _idx_start(c + 2, b)

            if s == 0:
                _idx_wait(o)
                _gathers_start(o)
            else:
                @pl.when(j < NPAIR - 1)
                def _():
                    _idx_wait(o)
                    _gathers_start(o)

            _compute(b)
            _scatter_start(b)
        return carry

    lax.fori_loop(0, NPAIR, _pair, 0)
    _scatter_wait(buf[0])
    _scatter_wait(buf[1])

    plsc.subcore_barrier()

    pltpu.sync_copy(acc_sp.at[pl.ds(sub * RPT, RPT)],
                    part_hbm.at[core, pl.ds(sub * RPT, RPT)])


def _edge_attention(q_tab, kv_tab, src, dst):
    mesh = plsc.VectorSubcoreMesh(core_axis_name="c", subcore_axis_name="s")
    return pl.kernel(
        _edge_body,
        out_type=jax.ShapeDtypeStruct((NC, NP, ROW), jnp.float32),
        mesh=mesh,
        compiler_params=pltpu.CompilerParams(
            use_tc_tiling_on_sc=False, needs_layout_passes=False),
        scratch_types=[
            pltpu.VMEM_SHARED((NP, ROW), jnp.float32),
            pltpu.VMEM((C,), jnp.int32),
            pltpu.VMEM((C,), jnp.int32),
            pltpu.VMEM((C,), jnp.int32),
            pltpu.VMEM((C,), jnp.int32),
            pltpu.VMEM((C,), jnp.int32),
            pltpu.VMEM((C,), jnp.int32),
            pltpu.VMEM((C, 2 * D), jnp.float32),
            pltpu.VMEM((C, 2 * D), jnp.float32),
            pltpu.VMEM((C, D), jnp.float32),
            pltpu.VMEM((C, D), jnp.float32),
            pltpu.VMEM((C, ROW), jnp.float32),
            pltpu.VMEM((C, ROW), jnp.float32),
        ] + [pltpu.SemaphoreType.DMA] * 8,
    )(q_tab, kv_tab, src, dst)


def _out_body(part_ref, wo_ref, o_ref):
    both = part_ref[...]
    tot = both[0] + both[1]
    wv = tot[:, :D]
    z = tot[:, D:D + H]
    rows = lax.broadcasted_iota(jnp.int32, (H, D), 0)
    cols = lax.broadcasted_iota(jnp.int32, (H, D), 1)
    sel = (cols // DK == rows).astype(jnp.float32)
    norm = jnp.dot(z, sel, preferred_element_type=jnp.float32) + 1e-6
    o_ref[...] = jnp.dot(wv / norm, wo_ref[...],
                         preferred_element_type=jnp.float32)


def _finalize(part, wo):
    blk = 1000
    return pl.pallas_call(
        _out_body,
        grid=(N // blk,),
        in_specs=[
            pl.BlockSpec((NC, blk, ROW), lambda i: (0, i, 0)),
            pl.BlockSpec((D, D), lambda i: (0, 0)),
        ],
        out_specs=pl.BlockSpec((blk, D), lambda i: (i, 0)),
        out_shape=jax.ShapeDtypeStruct((N, D), jnp.float32),
    )(part, wo)


@jax.jit
def kernel(x, edge_index, Wq, Wk, Wv, Wo):
    w = jnp.concatenate([Wq, Wk * 0.25, Wv], axis=1)
    q_tab, kv_tab = _project(x, w)
    src = edge_index[0].astype(jnp.int32)
    dst = edge_index[1].astype(jnp.int32)
    pad = EPAD - E
    src_p = jnp.concatenate([src, jnp.zeros((pad,), jnp.int32)])
    dst_p = jnp.concatenate([dst, jnp.full((pad,), TRASH, jnp.int32)])
    part = _edge_attention(q_tab, kv_tab, src_p, dst_p)
    return _finalize(part, Wo)
